# Initial kernel scaffold; baseline (speedup 1.0000x reference)
import jax
import jax.numpy as jnp
from jax.experimental import pallas as pl


def kernel(x, emb, W1, b1, W2, b2, W3, b3, Wo, bo):
    raise NotImplementedError("write your pallas kernel here")


# speedup vs baseline: 2.7769x; 2.7769x over previous
import functools

import jax
import jax.numpy as jnp
from jax import lax
from jax.experimental import pallas as pl
from jax.experimental.pallas import tpu as pltpu
from jax.experimental.pallas import tpu_sc as plsc

_D = 64
_NF = 26
_B = 4096
_ROWS = _NF * _B
_NW = 32
_RPW = _ROWS // _NW
_CH = 128
_NCH = _RPW // _CH

_DIN = _NF * _D
_BT = 512


def _sc_gather(idx3, table):
    mesh = plsc.VectorSubcoreMesh(core_axis_name="c", subcore_axis_name="s")

    @functools.partial(
        pl.kernel,
        mesh=mesh,
        compiler_params=pltpu.CompilerParams(use_tc_tiling_on_sc=False),
        out_type=jax.ShapeDtypeStruct((_ROWS, _D), jnp.float32),
        scratch_types=[
            pltpu.VMEM((_NCH, _CH), jnp.int32),
            pltpu.VMEM((2, _CH, _D), jnp.float32),
            pltpu.SemaphoreType.DMA,
            pltpu.SemaphoreType.DMA,
        ],
    )
    def gather_k(idx_hbm, table_hbm, out_hbm, idx_v, rows_v, sem0, sem1):
        wid = lax.axis_index("s") * 2 + lax.axis_index("c")
        base = wid * _RPW
        pltpu.sync_copy(idx_hbm.at[wid], idx_v)

        def fire(c, slot, sem):
            pltpu.async_copy(table_hbm.at[idx_v.at[c]], rows_v.at[slot], sem)

        def drain(slot, sem):
            pltpu.make_async_copy(
                table_hbm.at[pl.ds(0, _CH)], rows_v.at[slot], sem
            ).wait()

        fire(0, 0, sem0)

        def outer(o, carry):
            c0 = 2 * o
            fire(c0 + 1, 1, sem1)
            drain(0, sem0)
            pltpu.sync_copy(rows_v.at[0], out_hbm.at[pl.ds(base + c0 * _CH, _CH)])

            @pl.when(c0 + 2 < _NCH)
            def _():
                fire(c0 + 2, 0, sem0)

            drain(1, sem1)
            pltpu.sync_copy(
                rows_v.at[1], out_hbm.at[pl.ds(base + (c0 + 1) * _CH, _CH)]
            )
            return carry

        lax.fori_loop(0, _NCH // 2, outer, 0)

    return gather_k(idx3, table)


def _mlp_body(x_ref, w1_ref, b1_ref, w2_ref, b2_ref, w3_ref, b3_ref,
              wo_ref, bo_ref, o_ref):
    h = x_ref[...]
    h = jnp.maximum(
        jnp.dot(h, w1_ref[...], preferred_element_type=jnp.float32) + b1_ref[...],
        0.0)
    h = jnp.maximum(
        jnp.dot(h, w2_ref[...], preferred_element_type=jnp.float32) + b2_ref[...],
        0.0)
    h = jnp.maximum(
        jnp.dot(h, w3_ref[...], preferred_element_type=jnp.float32) + b3_ref[...],
        0.0)
    logit = jnp.sum(h * wo_ref[...], axis=1, keepdims=True) + bo_ref[...]
    o_ref[...] = jax.nn.sigmoid(logit)


def _tc_mlp(x2d, W1, b1, W2, b2, W3, b3, Wo, bo):
    d1, d2, d3 = W1.shape[1], W2.shape[1], W3.shape[1]
    rep = lambda shape: pl.BlockSpec(shape, lambda i: (0, 0))
    return pl.pallas_call(
        _mlp_body,
        grid=(_B // _BT,),
        in_specs=[
            pl.BlockSpec((_BT, _DIN), lambda i: (i, 0)),
            rep((_DIN, d1)), rep((1, d1)),
            rep((d1, d2)), rep((1, d2)),
            rep((d2, d3)), rep((1, d3)),
            rep((1, d3)), rep((1, 1)),
        ],
        out_specs=pl.BlockSpec((_BT, 1), lambda i: (i, 0)),
        out_shape=jax.ShapeDtypeStruct((_B, 1), jnp.float32),
    )(x2d, W1, b1.reshape(1, d1), W2, b2.reshape(1, d2),
      W3, b3.reshape(1, d3), Wo.reshape(1, d3), bo.reshape(1, 1))


def kernel(x, emb, W1, b1, W2, b2, W3, b3, Wo, bo):
    idx = x.astype(jnp.int32).T.reshape(_NW, _NCH, _CH)
    gathered = _sc_gather(idx, emb)
    x2d = gathered.reshape(_B, _DIN)
    return _tc_mlp(x2d, W1, b1, W2, b2, W3, b3, Wo, bo)
